# Initial kernel scaffold; baseline (speedup 1.0000x reference)
import jax
import jax.numpy as jnp
from jax.experimental import pallas as pl


def kernel(x, regime, regime_weights):
    raise NotImplementedError("write your pallas kernel here")


# speedup vs baseline: 1.9857x; 1.9857x over previous
import functools

import jax
import jax.numpy as jnp
import numpy as np
from jax import lax
from jax.experimental import pallas as pl
from jax.experimental.pallas import tpu as pltpu
from jax.experimental.pallas import tpu_sc as plsc

N_REGIMES_ = 3
N_COLS_ = 5
LANES_ = 16


def _take16(vec, idx):
    dnums = lax.GatherDimensionNumbers(
        offset_dims=(), collapsed_slice_dims=(0,), start_index_map=(0,)
    )
    return lax.gather(
        vec,
        idx[:, None],
        dnums,
        (1,),
        mode=lax.GatherScatterMode.PROMISE_IN_BOUNDS,
    )


@functools.lru_cache(maxsize=None)
def _build_sc_gather(batch: int):
    info = plsc.get_sparse_core_info()
    nc, ns = info.num_cores, info.num_subcores
    nw = nc * ns
    assert batch % (nw * LANES_) == 0
    b_per_w = batch // nw
    mesh = plsc.VectorSubcoreMesh(core_axis_name="c", subcore_axis_name="s")

    @functools.partial(
        pl.kernel,
        mesh=mesh,
        out_type=jax.ShapeDtypeStruct((batch * N_COLS_,), jnp.float32),
        scratch_types=[
            pltpu.VMEM((b_per_w,), jnp.int32),
            pltpu.VMEM((LANES_,), jnp.float32),
            pltpu.VMEM((b_per_w * N_COLS_,), jnp.float32),
        ],
    )
    def sc_gather(regime_hbm, table_hbm, out_hbm, idx_v, table_v, out_v):
        wid = lax.axis_index("s") * nc + lax.axis_index("c")
        base = wid * b_per_w
        pltpu.sync_copy(table_hbm, table_v)
        pltpu.sync_copy(regime_hbm.at[pl.ds(base, b_per_w)], idx_v)
        table_reg = table_v[...]
        zero = jnp.zeros((LANES_,), jnp.int32)
        top = jnp.full((LANES_,), N_REGIMES_ - 1, jnp.int32)
        lane = lax.iota(jnp.int32, LANES_)
        five = jnp.full((LANES_,), N_COLS_, jnp.int32)
        ioffs, jmods = [], []
        for p in range(N_COLS_):
            q = lane + p * LANES_
            ioffs.append(lax.div(q, five))
            jmods.append(lax.rem(q, five))
        for u in range(b_per_w // LANES_):
            r = idx_v[pl.ds(u * LANES_, LANES_)]
            r5 = jnp.minimum(jnp.maximum(r, zero), top) * N_COLS_
            for p in range(N_COLS_):
                gidx = _take16(r5, ioffs[p]) + jmods[p]
                out_v[pl.ds(u * LANES_ * N_COLS_ + p * LANES_, LANES_)] = _take16(
                    table_reg, gidx
                )
        pltpu.sync_copy(out_v, out_hbm.at[pl.ds(base * N_COLS_, b_per_w * N_COLS_)])

    return sc_gather


def kernel(x, regime, regime_weights):
    del x
    batch = regime.shape[0]
    regime = regime.astype(jnp.int32)
    table = jnp.zeros((LANES_,), jnp.float32).at[: N_REGIMES_ * N_COLS_].set(
        regime_weights.astype(jnp.float32).reshape(-1)
    )
    flat = _build_sc_gather(batch)(regime, table)
    return flat.reshape(batch, N_COLS_)


# speedup vs baseline: 2.0326x; 1.0237x over previous
import functools

import jax
import jax.numpy as jnp
import numpy as np
from jax import lax
from jax.experimental import pallas as pl
from jax.experimental.pallas import tpu as pltpu
from jax.experimental.pallas import tpu_sc as plsc

N_REGIMES_ = 3
N_COLS_ = 5
LANES_ = 16


def _take16(vec, idx):
    dnums = lax.GatherDimensionNumbers(
        offset_dims=(), collapsed_slice_dims=(0,), start_index_map=(0,)
    )
    return lax.gather(
        vec,
        idx[:, None],
        dnums,
        (1,),
        mode=lax.GatherScatterMode.PROMISE_IN_BOUNDS,
    )


@functools.lru_cache(maxsize=None)
def _build_sc_gather(batch: int):
    info = plsc.get_sparse_core_info()
    nc, ns = info.num_cores, info.num_subcores
    nw = nc * ns
    assert batch % (nw * LANES_) == 0
    b_per_w = batch // nw
    mesh = plsc.VectorSubcoreMesh(core_axis_name="c", subcore_axis_name="s")

    @functools.partial(
        pl.kernel,
        mesh=mesh,
        out_type=jax.ShapeDtypeStruct((batch * N_COLS_,), jnp.float32),
        scratch_types=[
            pltpu.VMEM((b_per_w,), jnp.int32),
            pltpu.VMEM((LANES_,), jnp.float32),
            pltpu.VMEM((b_per_w * N_COLS_,), jnp.float32),
        ],
    )
    def sc_gather(regime_hbm, table_hbm, out_hbm, idx_v, table_v, out_v):
        wid = lax.axis_index("s") * nc + lax.axis_index("c")
        base = wid * b_per_w
        pltpu.sync_copy(table_hbm, table_v.at[pl.ds(0, N_REGIMES_ * N_COLS_)])
        pltpu.sync_copy(regime_hbm.at[pl.ds(base, b_per_w)], idx_v)
        table_reg = table_v[...]
        zero = jnp.zeros((LANES_,), jnp.int32)
        top = jnp.full((LANES_,), N_REGIMES_ - 1, jnp.int32)
        lane = lax.iota(jnp.int32, LANES_)
        five = jnp.full((LANES_,), N_COLS_, jnp.int32)
        ioffs, jmods = [], []
        for p in range(N_COLS_):
            q = lane + p * LANES_
            ioffs.append(lax.div(q, five))
            jmods.append(lax.rem(q, five))
        for u in range(b_per_w // LANES_):
            r = idx_v[pl.ds(u * LANES_, LANES_)]
            r5 = jnp.minimum(jnp.maximum(r, zero), top) * N_COLS_
            for p in range(N_COLS_):
                gidx = _take16(r5, ioffs[p]) + jmods[p]
                out_v[pl.ds(u * LANES_ * N_COLS_ + p * LANES_, LANES_)] = _take16(
                    table_reg, gidx
                )
        pltpu.sync_copy(out_v, out_hbm.at[pl.ds(base * N_COLS_, b_per_w * N_COLS_)])

    return sc_gather


def kernel(x, regime, regime_weights):
    del x
    batch = regime.shape[0]
    regime = regime.astype(jnp.int32)
    table = regime_weights.astype(jnp.float32).reshape(-1)
    flat = _build_sc_gather(batch)(regime, table)
    return flat.reshape(batch, N_COLS_)


# speedup vs baseline: 3.6666x; 1.8039x over previous
import functools

import jax
import jax.numpy as jnp
from jax import lax
from jax.experimental import pallas as pl
from jax.experimental.pallas import tpu as pltpu
from jax.experimental.pallas import tpu_sc as plsc

N_REGIMES_ = 3
N_COLS_ = 5
LANES_ = 16


def _take16(vec, idx):
    dnums = lax.GatherDimensionNumbers(
        offset_dims=(), collapsed_slice_dims=(0,), start_index_map=(0,)
    )
    return lax.gather(
        vec,
        idx[:, None],
        dnums,
        (1,),
        mode=lax.GatherScatterMode.PROMISE_IN_BOUNDS,
    )


@functools.lru_cache(maxsize=None)
def _build_sc_gather(batch: int):
    info = plsc.get_sparse_core_info()
    nc, ns = info.num_cores, info.num_subcores
    nw = nc * ns
    assert batch % (nw * LANES_) == 0
    b_per_w = batch // nw
    mesh = plsc.VectorSubcoreMesh(core_axis_name="c", subcore_axis_name="s")

    @functools.partial(
        pl.kernel,
        mesh=mesh,
        out_type=jax.ShapeDtypeStruct((N_COLS_, batch), jnp.float32),
        scratch_types=[
            pltpu.VMEM((b_per_w,), jnp.int32),
            pltpu.VMEM((LANES_,), jnp.float32),
            pltpu.VMEM((N_COLS_, b_per_w), jnp.float32),
            pltpu.SemaphoreType.DMA,
        ],
    )
    def sc_gather(regime_hbm, table_hbm, out_hbm, idx_v, table_v, out_v, sem):
        wid = lax.axis_index("s") * nc + lax.axis_index("c")
        base = wid * b_per_w
        pltpu.sync_copy(table_hbm, table_v.at[pl.ds(0, N_REGIMES_ * N_COLS_)])
        pltpu.sync_copy(regime_hbm.at[pl.ds(base, b_per_w)], idx_v)
        table_reg = table_v[...]
        zero = jnp.zeros((LANES_,), jnp.int32)
        top = jnp.full((LANES_,), N_REGIMES_ - 1, jnp.int32)
        for u in range(b_per_w // LANES_):
            r = idx_v[pl.ds(u * LANES_, LANES_)]
            r5 = jnp.minimum(jnp.maximum(r, zero), top) * N_COLS_
            for j in range(N_COLS_):
                out_v[j, pl.ds(u * LANES_, LANES_)] = _take16(table_reg, r5 + j)
        pltpu.async_copy(
            out_v, out_hbm.at[:, pl.ds(base, b_per_w)], sem
        ).wait()

    return sc_gather


def kernel(x, regime, regime_weights):
    del x
    batch = regime.shape[0]
    regime = regime.astype(jnp.int32)
    table = regime_weights.astype(jnp.float32).reshape(-1)
    out_t = _build_sc_gather(batch)(regime, table)
    return out_t.T
